# baseline (speedup 1.0000x reference)
import jax
import jax.numpy as jnp
from jax.experimental import pallas as pl
from jax.experimental.pallas import tpu as pltpu

_BLOCK_B = 16384


def _tc_body(idx_ref, table_ref, o0_ref, o1_ref, cols, win, sem):
    i = pl.program_id(0)

    @pl.when(i == 0)
    def _fetch_cols():
        d = win.shape[1]
        cps = [
            pltpu.make_async_copy(
                table_ref.at[:, pl.ds((idx_ref[k] // 128) * 128, 128)],
                win.at[k], sem)
            for k in range(2)
        ]
        for cp in cps:
            cp.start()
        for cp in cps:
            cp.wait()
        lane = jax.lax.broadcasted_iota(jnp.int32, (d, 128), 1)
        for k in range(2):
            off = idx_ref[k] % 128
            colk = jnp.sum(
                jnp.where(lane == off, win[k], 0.0), axis=1, keepdims=True)
            cols[:, pl.ds(k, 1)] = colk

    o0_ref[...] = jnp.broadcast_to(cols[:, 0:1], o0_ref.shape)
    o1_ref[...] = jnp.broadcast_to(cols[:, 1:2], o1_ref.shape)


def kernel(rel, table, indices):
    batch = rel.shape[0]
    d = table.shape[1]
    table_t = table.T
    grid = (batch // _BLOCK_B,)
    out_sds = jax.ShapeDtypeStruct((d, batch), jnp.float32)
    o0, o1 = pl.pallas_call(
        _tc_body,
        grid_spec=pltpu.PrefetchScalarGridSpec(
            num_scalar_prefetch=1,
            grid=grid,
            in_specs=[pl.BlockSpec(memory_space=pl.ANY)],
            out_specs=[
                pl.BlockSpec((d, _BLOCK_B), lambda i, idx: (0, i)),
                pl.BlockSpec((d, _BLOCK_B), lambda i, idx: (0, i)),
            ],
            scratch_shapes=[
                pltpu.VMEM((d, 2), jnp.float32),
                pltpu.VMEM((2, d, 128), jnp.float32),
                pltpu.SemaphoreType.DMA,
            ],
        ),
        out_shape=[out_sds, out_sds],
    )(indices.astype(jnp.int32), table_t)
    return (o0.T, o1.T)


# speedup vs baseline: 1.0355x; 1.0355x over previous
import jax
import jax.numpy as jnp
from jax.experimental import pallas as pl
from jax.experimental.pallas import tpu as pltpu

_BLOCK_B = 8192


def _tc_body(idx_ref, table_ref, o0_ref, o1_ref, cols, win, sem):
    i = pl.program_id(0)

    @pl.when(i == 0)
    def _fetch_cols():
        d = win.shape[1]
        cps = [
            pltpu.make_async_copy(
                table_ref.at[:, pl.ds((idx_ref[k] // 128) * 128, 128)],
                win.at[k], sem)
            for k in range(2)
        ]
        for cp in cps:
            cp.start()
        for cp in cps:
            cp.wait()
        lane = jax.lax.broadcasted_iota(jnp.int32, (d, 128), 1)
        for k in range(2):
            off = idx_ref[k] % 128
            colk = jnp.sum(
                jnp.where(lane == off, win[k], 0.0), axis=1, keepdims=True)
            cols[:, pl.ds(k, 1)] = colk

    o0_ref[...] = jnp.broadcast_to(cols[:, 0:1], o0_ref.shape)
    o1_ref[...] = jnp.broadcast_to(cols[:, 1:2], o1_ref.shape)


def kernel(rel, table, indices):
    batch = rel.shape[0]
    d = table.shape[1]
    table_t = table.T
    grid = (batch // _BLOCK_B,)
    out_sds = jax.ShapeDtypeStruct((d, batch), jnp.float32)
    o0, o1 = pl.pallas_call(
        _tc_body,
        grid_spec=pltpu.PrefetchScalarGridSpec(
            num_scalar_prefetch=1,
            grid=grid,
            in_specs=[pl.BlockSpec(memory_space=pl.ANY)],
            out_specs=[
                pl.BlockSpec((d, _BLOCK_B), lambda i, idx: (0, i)),
                pl.BlockSpec((d, _BLOCK_B), lambda i, idx: (0, i)),
            ],
            scratch_shapes=[
                pltpu.VMEM((d, 2), jnp.float32),
                pltpu.VMEM((2, d, 128), jnp.float32),
                pltpu.SemaphoreType.DMA,
            ],
        ),
        out_shape=[out_sds, out_sds],
    )(indices.astype(jnp.int32), table_t)
    return (o0.T, o1.T)
